# baseline (device time: 839736 ns/iter reference)
import jax
import jax.numpy as jnp
from jax import lax
from jax.experimental import pallas as pl
from jax.experimental.pallas import tpu as pltpu

N_DEV = 16
M = 4096
N = 8192
CW = N // N_DEV
CW2 = CW // 2
HROW = M // 2
N_SUB = 2
SROW = HROW // N_SUB
N_LANE = 2 * N_SUB


class _Lane:
    def __init__(self, me, sgn, half, rsub, to, frm, bufs, sems, creds):
        self.me, self.sgn, self.half, self.rsub = me, sgn, half, rsub
        self.to, self.frm = to, frm
        self.send, self.recv, self.ld, self.acc, self.stg = bufs
        (self.ssem, self.rsem, self.ldsem, self.stsem,
         self.agss, self.agrs) = sems
        self.rscred, self.agcred = creds
        self.rdma = None

    def chunk(self, k):
        return lax.rem(self.me + self.sgn * k + 2 * N_DEV, N_DEV)


def _ar_body(part_ref, scale_ref, out_ref, *scr):
    me = lax.axis_index("i")
    right = lax.rem(me + 1, N_DEV)
    left = lax.rem(me + N_DEV - 1, N_DEV)
    ro = pl.program_id(0) * HROW

    barrier = pltpu.get_barrier_semaphore()
    for nbr in (left, right):
        pl.semaphore_signal(barrier, inc=1, device_id=(nbr,),
                            device_id_type=pl.DeviceIdType.MESH)
    pl.semaphore_wait(barrier, 2)

    cfg = tuple((sgn, half, rsub)
                for rsub in range(N_SUB)
                for sgn, half in ((-1, 0), (+1, 1)))
    nb, ns = 5 * N_LANE, 6 * N_LANE
    lanes = tuple(
        _Lane(me, sgn, half, rsub,
              right if sgn < 0 else left,
              left if sgn < 0 else right,
              scr[5 * i: 5 * i + 5],
              scr[nb + 6 * i: nb + 6 * i + 6],
              scr[nb + ns + 2 * i: nb + ns + 2 * i + 2])
        for i, (sgn, half, rsub) in enumerate(cfg)
    )

    def blk(c, ln):
        return part_ref.at[pl.ds(ro + ln.rsub * SROW, SROW),
                           pl.ds(c * CW + ln.half * CW2, CW2)]

    def oblk(c, ln):
        return out_ref.at[pl.ds(ro + ln.rsub * SROW, SROW),
                          pl.ds(c * CW + ln.half * CW2, CW2)]

    def rs_rdma(ln, s):
        return pltpu.make_async_remote_copy(
            src_ref=ln.send.at[s % 2],
            dst_ref=ln.recv.at[s % 2],
            send_sem=ln.ssem.at[s % 2],
            recv_sem=ln.rsem.at[s % 2],
            device_id=(ln.to,),
            device_id_type=pl.DeviceIdType.MESH,
        )

    def ag_rdma(ln, t):
        return pltpu.make_async_remote_copy(
            src_ref=ln.send.at[t % 2],
            dst_ref=ln.send.at[(t + 1) % 2],
            send_sem=ln.agss.at[t % 2],
            recv_sem=ln.agrs.at[(t + 1) % 2],
            device_id=(ln.to,),
            device_id_type=pl.DeviceIdType.MESH,
        )

    for ln in lanes:
        pltpu.make_async_copy(blk(me, ln), ln.ld, ln.ldsem).start()
    for ln in lanes:
        pltpu.make_async_copy(blk(me, ln), ln.ld, ln.ldsem).wait()
        ln.send[0] = ln.ld[...].astype(jnp.bfloat16)
        ln.rdma = rs_rdma(ln, 0)
        ln.rdma.start()
        pltpu.make_async_copy(blk(ln.chunk(1), ln), ln.ld, ln.ldsem).start()

    for s in range(N_DEV - 1):
        slot = s % 2
        for ln in lanes:
            pltpu.make_async_copy(blk(ln.chunk(s + 1), ln),
                                  ln.ld, ln.ldsem).wait()
            ln.rdma.wait()
            if s < N_DEV - 2:
                ln.send[(s + 1) % 2] = (
                    ln.recv[slot].astype(jnp.float32) + ln.ld[...]
                ).astype(jnp.bfloat16)
            else:
                ln.acc[...] = ln.recv[slot].astype(jnp.float32) + ln.ld[...]
            if s <= N_DEV - 4:
                pl.semaphore_signal(ln.rscred, inc=1, device_id=(ln.frm,),
                                    device_id_type=pl.DeviceIdType.MESH)
            if s == N_DEV - 3:
                pl.semaphore_signal(ln.agcred, inc=1, device_id=(ln.frm,),
                                    device_id_type=pl.DeviceIdType.MESH)
            if s < N_DEV - 2:
                if s + 1 >= 2:
                    pl.semaphore_wait(ln.rscred, 1)
                ln.rdma = rs_rdma(ln, s + 1)
                ln.rdma.start()
                pltpu.make_async_copy(blk(ln.chunk(s + 2), ln),
                                      ln.ld, ln.ldsem).start()

    for ln in lanes:
        own = ln.chunk(N_DEV - 1)
        y = ln.acc[...] * scale_ref[0, 0]
        sg = 1.0 / (1.0 + jnp.exp(-jnp.clip(y, -60.0, 60.0)))
        ln.stg[1] = y * sg
        ln.send[0] = ln.stg[1].astype(jnp.bfloat16)
        pltpu.make_async_copy(ln.stg.at[1], oblk(own, ln),
                              ln.stsem.at[1]).start()
        pl.semaphore_wait(ln.agcred, 1)
        ln.rdma = ag_rdma(ln, 0)
        ln.rdma.start()

    for t in range(N_DEV - 1):
        for ln in lanes:
            ln.rdma.wait()
            if t <= N_DEV - 3:
                pl.semaphore_signal(ln.agcred, inc=1, device_id=(ln.frm,),
                                    device_id_type=pl.DeviceIdType.MESH)
            if t < N_DEV - 2:
                pl.semaphore_wait(ln.agcred, 1)
                ln.rdma = ag_rdma(ln, t + 1)
                ln.rdma.start()
            r = ln.chunk(t)
            if t >= 1:
                pltpu.make_async_copy(ln.stg.at[t % 2], oblk(r, ln),
                                      ln.stsem.at[t % 2]).wait()
            ln.stg[t % 2] = ln.send[(t + 1) % 2].astype(jnp.float32)
            pltpu.make_async_copy(ln.stg.at[t % 2], oblk(r, ln),
                                  ln.stsem.at[t % 2]).start()
    for ln in lanes:
        for sl in (1, 0):
            pltpu.make_async_copy(ln.stg.at[sl], oblk(me, ln),
                                  ln.stsem.at[sl]).wait()


def kernel(x, w_mat, scale_x, scale_w):
    partial = jnp.dot(x.astype(jnp.bfloat16), w_mat.astype(jnp.bfloat16),
                      preferred_element_type=jnp.float32)
    scale = jnp.reshape(scale_x.astype(jnp.float32)
                        * scale_w.astype(jnp.float32), (1, 1))

    lane_bufs = [
        pltpu.VMEM((2, SROW, CW2), jnp.bfloat16),
        pltpu.VMEM((2, SROW, CW2), jnp.bfloat16),
        pltpu.VMEM((SROW, CW2), jnp.float32),
        pltpu.VMEM((SROW, CW2), jnp.float32),
        pltpu.VMEM((2, SROW, CW2), jnp.float32),
    ]
    lane_sems = [
        pltpu.SemaphoreType.DMA((2,)),
        pltpu.SemaphoreType.DMA((2,)),
        pltpu.SemaphoreType.DMA,
        pltpu.SemaphoreType.DMA((2,)),
        pltpu.SemaphoreType.DMA((2,)),
        pltpu.SemaphoreType.DMA((2,)),
    ]
    scratch = (lane_bufs * N_LANE
               + lane_sems * N_LANE
               + [pltpu.SemaphoreType.REGULAR] * (2 * N_LANE))

    return pl.pallas_call(
        _ar_body,
        grid=(2,),
        out_shape=jax.ShapeDtypeStruct((M, N), jnp.float32),
        in_specs=[
            pl.BlockSpec(memory_space=pl.ANY),
            pl.BlockSpec(memory_space=pltpu.MemorySpace.SMEM),
        ],
        out_specs=pl.BlockSpec(memory_space=pl.ANY),
        scratch_shapes=scratch,
        compiler_params=pltpu.CompilerParams(collective_id=0),
    )(partial, scale)


# device time: 829870 ns/iter; 1.0119x vs baseline; 1.0119x over previous
import jax
import jax.numpy as jnp
from jax import lax
from jax.experimental import pallas as pl
from jax.experimental.pallas import tpu as pltpu

N_DEV = 16
M = 4096
N = 8192
CW = N // N_DEV
CW2 = CW // 2
HROW = M // 2
N_SUB = 2
SROW = HROW // N_SUB
N_LANE = 2 * N_SUB

_RING_COORDS = [(0, 0, 0), (0, 0, 1), (0, 0, 2), (0, 0, 3),
                (0, 1, 3), (0, 1, 2), (0, 1, 1), (0, 1, 0),
                (1, 1, 0), (1, 1, 1), (1, 1, 2), (1, 1, 3),
                (1, 0, 3), (1, 0, 2), (1, 0, 1), (1, 0, 0)]
_PLANE_IDX = {(0, 0): 0, (1, 0): 1, (1, 1): 2, (0, 1): 3}
PERM = tuple(4 * z + _PLANE_IDX[(x, y)] for x, y, z in _RING_COORDS)
INV = tuple(PERM.index(m) for m in range(N_DEV))
NEXTDEV = tuple(PERM[(INV[m] + 1) % N_DEV] for m in range(N_DEV))
PREVDEV = tuple(PERM[(INV[m] - 1) % N_DEV] for m in range(N_DEV))


class _Lane:
    def __init__(self, me, sgn, half, rsub, to, frm, bufs, sems, creds):
        self.me, self.sgn, self.half, self.rsub = me, sgn, half, rsub
        self.to, self.frm = to, frm
        self.send, self.recv, self.ld, self.acc, self.stg = bufs
        (self.ssem, self.rsem, self.ldsem, self.stsem,
         self.agss, self.agrs) = sems
        self.rscred, self.agcred = creds
        self.rdma = None

    def chunk(self, k):
        return lax.rem(self.me + self.sgn * k + 2 * N_DEV, N_DEV)


def _ar_body(part_ref, scale_ref, out_ref, *scr):
    me_pos = lax.axis_index("i")

    def lut(table):
        v = jnp.int32(0)
        for m in range(N_DEV):
            v = v + jnp.int32(table[m]) * (me_pos == m).astype(jnp.int32)
        return v

    me = lut(INV)
    right = lut(NEXTDEV)
    left = lut(PREVDEV)
    ro = pl.program_id(0) * HROW

    barrier = pltpu.get_barrier_semaphore()
    for nbr in (left, right):
        pl.semaphore_signal(barrier, inc=1, device_id=(nbr,),
                            device_id_type=pl.DeviceIdType.MESH)
    pl.semaphore_wait(barrier, 2)

    cfg = tuple((sgn, half, rsub)
                for rsub in range(N_SUB)
                for sgn, half in ((-1, 0), (+1, 1)))
    nb, ns = 5 * N_LANE, 6 * N_LANE
    lanes = tuple(
        _Lane(me, sgn, half, rsub,
              right if sgn < 0 else left,
              left if sgn < 0 else right,
              scr[5 * i: 5 * i + 5],
              scr[nb + 6 * i: nb + 6 * i + 6],
              scr[nb + ns + 2 * i: nb + ns + 2 * i + 2])
        for i, (sgn, half, rsub) in enumerate(cfg)
    )

    def blk(c, ln):
        return part_ref.at[pl.ds(ro + ln.rsub * SROW, SROW),
                           pl.ds(c * CW + ln.half * CW2, CW2)]

    def oblk(c, ln):
        return out_ref.at[pl.ds(ro + ln.rsub * SROW, SROW),
                          pl.ds(c * CW + ln.half * CW2, CW2)]

    def rs_rdma(ln, s):
        return pltpu.make_async_remote_copy(
            src_ref=ln.send.at[s % 2],
            dst_ref=ln.recv.at[s % 2],
            send_sem=ln.ssem.at[s % 2],
            recv_sem=ln.rsem.at[s % 2],
            device_id=(ln.to,),
            device_id_type=pl.DeviceIdType.MESH,
        )

    def ag_rdma(ln, t):
        return pltpu.make_async_remote_copy(
            src_ref=ln.send.at[t % 2],
            dst_ref=ln.send.at[(t + 1) % 2],
            send_sem=ln.agss.at[t % 2],
            recv_sem=ln.agrs.at[(t + 1) % 2],
            device_id=(ln.to,),
            device_id_type=pl.DeviceIdType.MESH,
        )

    for ln in lanes:
        pltpu.make_async_copy(blk(me, ln), ln.ld, ln.ldsem).start()
    for ln in lanes:
        pltpu.make_async_copy(blk(me, ln), ln.ld, ln.ldsem).wait()
        ln.send[0] = ln.ld[...].astype(jnp.bfloat16)
        ln.rdma = rs_rdma(ln, 0)
        ln.rdma.start()
        pltpu.make_async_copy(blk(ln.chunk(1), ln), ln.ld, ln.ldsem).start()

    for s in range(N_DEV - 1):
        slot = s % 2
        for ln in lanes:
            pltpu.make_async_copy(blk(ln.chunk(s + 1), ln),
                                  ln.ld, ln.ldsem).wait()
            ln.rdma.wait()
            if s < N_DEV - 2:
                ln.send[(s + 1) % 2] = (
                    ln.recv[slot].astype(jnp.float32) + ln.ld[...]
                ).astype(jnp.bfloat16)
            else:
                ln.acc[...] = ln.recv[slot].astype(jnp.float32) + ln.ld[...]
            if s <= N_DEV - 4:
                pl.semaphore_signal(ln.rscred, inc=1, device_id=(ln.frm,),
                                    device_id_type=pl.DeviceIdType.MESH)
            if s == N_DEV - 3:
                pl.semaphore_signal(ln.agcred, inc=1, device_id=(ln.frm,),
                                    device_id_type=pl.DeviceIdType.MESH)
            if s < N_DEV - 2:
                if s + 1 >= 2:
                    pl.semaphore_wait(ln.rscred, 1)
                ln.rdma = rs_rdma(ln, s + 1)
                ln.rdma.start()
                pltpu.make_async_copy(blk(ln.chunk(s + 2), ln),
                                      ln.ld, ln.ldsem).start()

    for ln in lanes:
        own = ln.chunk(N_DEV - 1)
        y = ln.acc[...] * scale_ref[0, 0]
        sg = 1.0 / (1.0 + jnp.exp(-jnp.clip(y, -60.0, 60.0)))
        ln.stg[1] = y * sg
        ln.send[0] = ln.stg[1].astype(jnp.bfloat16)
        pltpu.make_async_copy(ln.stg.at[1], oblk(own, ln),
                              ln.stsem.at[1]).start()
        pl.semaphore_wait(ln.agcred, 1)
        ln.rdma = ag_rdma(ln, 0)
        ln.rdma.start()

    for t in range(N_DEV - 1):
        for ln in lanes:
            ln.rdma.wait()
            if t <= N_DEV - 3:
                pl.semaphore_signal(ln.agcred, inc=1, device_id=(ln.frm,),
                                    device_id_type=pl.DeviceIdType.MESH)
            if t < N_DEV - 2:
                pl.semaphore_wait(ln.agcred, 1)
                ln.rdma = ag_rdma(ln, t + 1)
                ln.rdma.start()
            r = ln.chunk(t)
            if t >= 1:
                pltpu.make_async_copy(ln.stg.at[t % 2], oblk(r, ln),
                                      ln.stsem.at[t % 2]).wait()
            ln.stg[t % 2] = ln.send[(t + 1) % 2].astype(jnp.float32)
            pltpu.make_async_copy(ln.stg.at[t % 2], oblk(r, ln),
                                  ln.stsem.at[t % 2]).start()
    for ln in lanes:
        for sl in (1, 0):
            pltpu.make_async_copy(ln.stg.at[sl], oblk(me, ln),
                                  ln.stsem.at[sl]).wait()


def kernel(x, w_mat, scale_x, scale_w):
    partial = jnp.dot(x.astype(jnp.bfloat16), w_mat.astype(jnp.bfloat16),
                      preferred_element_type=jnp.float32)
    scale = jnp.reshape(scale_x.astype(jnp.float32)
                        * scale_w.astype(jnp.float32), (1, 1))

    lane_bufs = [
        pltpu.VMEM((2, SROW, CW2), jnp.bfloat16),
        pltpu.VMEM((2, SROW, CW2), jnp.bfloat16),
        pltpu.VMEM((SROW, CW2), jnp.float32),
        pltpu.VMEM((SROW, CW2), jnp.float32),
        pltpu.VMEM((2, SROW, CW2), jnp.float32),
    ]
    lane_sems = [
        pltpu.SemaphoreType.DMA((2,)),
        pltpu.SemaphoreType.DMA((2,)),
        pltpu.SemaphoreType.DMA,
        pltpu.SemaphoreType.DMA((2,)),
        pltpu.SemaphoreType.DMA((2,)),
        pltpu.SemaphoreType.DMA((2,)),
    ]
    scratch = (lane_bufs * N_LANE
               + lane_sems * N_LANE
               + [pltpu.SemaphoreType.REGULAR] * (2 * N_LANE))

    return pl.pallas_call(
        _ar_body,
        grid=(2,),
        out_shape=jax.ShapeDtypeStruct((M, N), jnp.float32),
        in_specs=[
            pl.BlockSpec(memory_space=pl.ANY),
            pl.BlockSpec(memory_space=pltpu.MemorySpace.SMEM),
        ],
        out_specs=pl.BlockSpec(memory_space=pl.ANY),
        scratch_shapes=scratch,
        compiler_params=pltpu.CompilerParams(collective_id=0),
    )(partial, scale)


# device time: 824750 ns/iter; 1.0182x vs baseline; 1.0062x over previous
import jax
import jax.numpy as jnp
from jax import lax
from jax.experimental import pallas as pl
from jax.experimental.pallas import tpu as pltpu

N_DEV = 16
M = 4096
N = 8192
CW = N // N_DEV
CW2 = CW // 2
HROW = M // 2
N_SUB = 2
SROW = HROW // N_SUB
N_LANE = 2 * N_SUB

_RING_COORDS = [(0, 0, 0), (0, 0, 1), (0, 0, 2), (0, 0, 3),
                (0, 1, 3), (0, 1, 2), (0, 1, 1), (0, 1, 0),
                (1, 1, 0), (1, 1, 1), (1, 1, 2), (1, 1, 3),
                (1, 0, 3), (1, 0, 2), (1, 0, 1), (1, 0, 0)]
_PLANE_IDX = {(0, 0): 0, (1, 0): 1, (1, 1): 2, (0, 1): 3}
PERM = tuple(4 * z + _PLANE_IDX[(x, y)] for x, y, z in _RING_COORDS)
INV = tuple(PERM.index(m) for m in range(N_DEV))
NEXTDEV = tuple(PERM[(INV[m] + 1) % N_DEV] for m in range(N_DEV))
PREVDEV = tuple(PERM[(INV[m] - 1) % N_DEV] for m in range(N_DEV))


class _Lane:
    def __init__(self, me, sgn, half, rsub, to, frm, bufs, sems, creds):
        self.me, self.sgn, self.half, self.rsub = me, sgn, half, rsub
        self.to, self.frm = to, frm
        self.send, self.recv, self.ld, self.stg = bufs
        (self.ssem, self.rsem, self.ldsem, self.stsem,
         self.agss, self.agrs) = sems
        self.rscred, self.agcred = creds
        self.rdma = None

    def chunk(self, k):
        return lax.rem(self.me + self.sgn * k + 2 * N_DEV, N_DEV)


def _ar_body(part_ref, scale_ref, out_ref, *scr):
    me_pos = lax.axis_index("i")

    def lut(table):
        v = jnp.int32(0)
        for m in range(N_DEV):
            v = v + jnp.int32(table[m]) * (me_pos == m).astype(jnp.int32)
        return v

    me = lut(INV)
    right = lut(NEXTDEV)
    left = lut(PREVDEV)
    ro = pl.program_id(0) * HROW

    barrier = pltpu.get_barrier_semaphore()
    for nbr in (left, right):
        pl.semaphore_signal(barrier, inc=1, device_id=(nbr,),
                            device_id_type=pl.DeviceIdType.MESH)
    pl.semaphore_wait(barrier, 2)

    cfg = tuple((sgn, half, rsub)
                for rsub in range(N_SUB)
                for sgn, half in ((-1, 0), (+1, 1)))
    nb, ns = 4 * N_LANE, 6 * N_LANE
    lanes = tuple(
        _Lane(me, sgn, half, rsub,
              right if sgn < 0 else left,
              left if sgn < 0 else right,
              scr[4 * i: 4 * i + 4],
              scr[nb + 6 * i: nb + 6 * i + 6],
              scr[nb + ns + 2 * i: nb + ns + 2 * i + 2])
        for i, (sgn, half, rsub) in enumerate(cfg)
    )

    def blk(c, ln):
        return part_ref.at[pl.ds(ro + ln.rsub * SROW, SROW),
                           pl.ds(c * CW + ln.half * CW2, CW2)]

    def oblk(c, ln):
        return out_ref.at[pl.ds(ro + ln.rsub * SROW, SROW),
                          pl.ds(c * CW + ln.half * CW2, CW2)]

    def rs_rdma(ln, s):
        return pltpu.make_async_remote_copy(
            src_ref=ln.send.at[s % 2],
            dst_ref=ln.recv.at[s % 2],
            send_sem=ln.ssem.at[s % 2],
            recv_sem=ln.rsem.at[s % 2],
            device_id=(ln.to,),
            device_id_type=pl.DeviceIdType.MESH,
        )

    def ag_rdma(ln, t):
        return pltpu.make_async_remote_copy(
            src_ref=ln.send.at[t % 2],
            dst_ref=ln.send.at[(t + 1) % 2],
            send_sem=ln.agss.at[t % 2],
            recv_sem=ln.agrs.at[(t + 1) % 2],
            device_id=(ln.to,),
            device_id_type=pl.DeviceIdType.MESH,
        )

    for ln in lanes:
        pltpu.make_async_copy(blk(me, ln), ln.ld, ln.ldsem).start()
    for ln in lanes:
        pltpu.make_async_copy(blk(me, ln), ln.ld, ln.ldsem).wait()
        ln.send[0] = ln.ld[...].astype(jnp.bfloat16)
        ln.rdma = rs_rdma(ln, 0)
        ln.rdma.start()
        pltpu.make_async_copy(blk(ln.chunk(1), ln), ln.ld, ln.ldsem).start()

    for s in range(N_DEV - 1):
        slot = s % 2
        for ln in lanes:
            pltpu.make_async_copy(blk(ln.chunk(s + 1), ln),
                                  ln.ld, ln.ldsem).wait()
            ln.rdma.wait()
            if s < N_DEV - 2:
                ln.send[(s + 1) % 2] = (
                    ln.recv[slot].astype(jnp.float32) + ln.ld[...]
                ).astype(jnp.bfloat16)
            else:
                own = ln.chunk(N_DEV - 1)
                y = (ln.recv[slot].astype(jnp.float32) + ln.ld[...]
                     ) * scale_ref[0, 0]
                sg = 1.0 / (1.0 + jnp.exp(-jnp.clip(y, -60.0, 60.0)))
                ln.stg[1] = y * sg
                ln.send[0] = ln.stg[1].astype(jnp.bfloat16)
                pltpu.make_async_copy(ln.stg.at[1], oblk(own, ln),
                                      ln.stsem.at[1]).start()
                pl.semaphore_wait(ln.agcred, 1)
                ln.rdma = ag_rdma(ln, 0)
                ln.rdma.start()
            if s <= N_DEV - 4:
                pl.semaphore_signal(ln.rscred, inc=1, device_id=(ln.frm,),
                                    device_id_type=pl.DeviceIdType.MESH)
            if s == N_DEV - 3:
                pl.semaphore_signal(ln.agcred, inc=1, device_id=(ln.frm,),
                                    device_id_type=pl.DeviceIdType.MESH)
            if s < N_DEV - 2:
                if s + 1 >= 2:
                    pl.semaphore_wait(ln.rscred, 1)
                ln.rdma = rs_rdma(ln, s + 1)
                ln.rdma.start()
                pltpu.make_async_copy(blk(ln.chunk(s + 2), ln),
                                      ln.ld, ln.ldsem).start()

    for t in range(N_DEV - 1):
        for ln in lanes:
            ln.rdma.wait()
            if t <= N_DEV - 3:
                pl.semaphore_signal(ln.agcred, inc=1, device_id=(ln.frm,),
                                    device_id_type=pl.DeviceIdType.MESH)
            if t < N_DEV - 2:
                pl.semaphore_wait(ln.agcred, 1)
                ln.rdma = ag_rdma(ln, t + 1)
                ln.rdma.start()
            r = ln.chunk(t)
            if t >= 1:
                pltpu.make_async_copy(ln.stg.at[t % 2], oblk(r, ln),
                                      ln.stsem.at[t % 2]).wait()
            ln.stg[t % 2] = ln.send[(t + 1) % 2].astype(jnp.float32)
            pltpu.make_async_copy(ln.stg.at[t % 2], oblk(r, ln),
                                  ln.stsem.at[t % 2]).start()
    for ln in lanes:
        for sl in (1, 0):
            pltpu.make_async_copy(ln.stg.at[sl], oblk(me, ln),
                                  ln.stsem.at[sl]).wait()


def kernel(x, w_mat, scale_x, scale_w):
    partial = jnp.dot(x.astype(jnp.bfloat16), w_mat.astype(jnp.bfloat16),
                      preferred_element_type=jnp.float32)
    scale = jnp.reshape(scale_x.astype(jnp.float32)
                        * scale_w.astype(jnp.float32), (1, 1))

    lane_bufs = [
        pltpu.VMEM((2, SROW, CW2), jnp.bfloat16),
        pltpu.VMEM((2, SROW, CW2), jnp.bfloat16),
        pltpu.VMEM((SROW, CW2), jnp.float32),
        pltpu.VMEM((2, SROW, CW2), jnp.float32),
    ]
    lane_sems = [
        pltpu.SemaphoreType.DMA((2,)),
        pltpu.SemaphoreType.DMA((2,)),
        pltpu.SemaphoreType.DMA,
        pltpu.SemaphoreType.DMA((2,)),
        pltpu.SemaphoreType.DMA((2,)),
        pltpu.SemaphoreType.DMA((2,)),
    ]
    scratch = (lane_bufs * N_LANE
               + lane_sems * N_LANE
               + [pltpu.SemaphoreType.REGULAR] * (2 * N_LANE))

    return pl.pallas_call(
        _ar_body,
        grid=(2,),
        out_shape=jax.ShapeDtypeStruct((M, N), jnp.float32),
        in_specs=[
            pl.BlockSpec(memory_space=pl.ANY),
            pl.BlockSpec(memory_space=pltpu.MemorySpace.SMEM),
        ],
        out_specs=pl.BlockSpec(memory_space=pl.ANY),
        scratch_shapes=scratch,
        compiler_params=pltpu.CompilerParams(collective_id=0),
    )(partial, scale)


# device time: 821945 ns/iter; 1.0216x vs baseline; 1.0034x over previous
import jax
import jax.numpy as jnp
from jax import lax
from jax.experimental import pallas as pl
from jax.experimental.pallas import tpu as pltpu

N_DEV = 16
M = 4096
N = 8192
CW = N // N_DEV
CW2 = CW // 2
HROW = M // 2
N_SUB = 2
SROW = HROW // N_SUB
N_LANE = 2 * N_SUB

_RING_COORDS = [(0, 0, 0), (0, 0, 1), (0, 0, 2), (0, 0, 3),
                (0, 1, 3), (0, 1, 2), (0, 1, 1), (0, 1, 0),
                (1, 1, 0), (1, 1, 1), (1, 1, 2), (1, 1, 3),
                (1, 0, 3), (1, 0, 2), (1, 0, 1), (1, 0, 0)]
_PLANE_IDX = {(0, 0): 0, (1, 0): 1, (1, 1): 2, (0, 1): 3}
PERM = tuple(4 * z + _PLANE_IDX[(x, y)] for x, y, z in _RING_COORDS)
INV = tuple(PERM.index(m) for m in range(N_DEV))
NEXTDEV = tuple(PERM[(INV[m] + 1) % N_DEV] for m in range(N_DEV))
PREVDEV = tuple(PERM[(INV[m] - 1) % N_DEV] for m in range(N_DEV))


class _Lane:
    def __init__(self, me, sgn, half, rsub, to, frm, bufs, sems, creds):
        self.me, self.sgn, self.half, self.rsub = me, sgn, half, rsub
        self.to, self.frm = to, frm
        self.send, self.recv, self.ld, self.stg = bufs
        (self.ssem, self.rsem, self.ldsem, self.stsem,
         self.agss, self.agrs) = sems
        self.rscred, self.agcred = creds
        self.rdma = None

    def chunk(self, k):
        return lax.rem(self.me + self.sgn * k + 2 * N_DEV, N_DEV)


def _ar_body(part_ref, scale_ref, out_ref, *scr):
    me_pos = lax.axis_index("i")

    def lut(table):
        v = jnp.int32(0)
        for m in range(N_DEV):
            v = v + jnp.int32(table[m]) * (me_pos == m).astype(jnp.int32)
        return v

    me = lut(INV)
    right = lut(NEXTDEV)
    left = lut(PREVDEV)

    barrier = pltpu.get_barrier_semaphore()
    for nbr in (left, right):
        pl.semaphore_signal(barrier, inc=1, device_id=(nbr,),
                            device_id_type=pl.DeviceIdType.MESH)
    pl.semaphore_wait(barrier, 2)

    cfg = tuple((sgn, half, rsub)
                for rsub in range(N_SUB)
                for sgn, half in ((-1, 0), (+1, 1)))
    nb, ns = 4 * N_LANE, 6 * N_LANE
    lanes = tuple(
        _Lane(me, sgn, half, rsub,
              right if sgn < 0 else left,
              left if sgn < 0 else right,
              scr[4 * i: 4 * i + 4],
              scr[nb + 6 * i: nb + 6 * i + 6],
              scr[nb + ns + 2 * i: nb + ns + 2 * i + 2])
        for i, (sgn, half, rsub) in enumerate(cfg)
    )

    def rs_rdma(ln, s):
        return pltpu.make_async_remote_copy(
            src_ref=ln.send.at[s % 2],
            dst_ref=ln.recv.at[s % 2],
            send_sem=ln.ssem.at[s % 2],
            recv_sem=ln.rsem.at[s % 2],
            device_id=(ln.to,),
            device_id_type=pl.DeviceIdType.MESH,
        )

    def ag_rdma(ln, t):
        return pltpu.make_async_remote_copy(
            src_ref=ln.send.at[t % 2],
            dst_ref=ln.send.at[(t + 1) % 2],
            send_sem=ln.agss.at[t % 2],
            recv_sem=ln.agrs.at[(t + 1) % 2],
            device_id=(ln.to,),
            device_id_type=pl.DeviceIdType.MESH,
        )

    def run_pass(ro):
        def blk(c, ln):
            return part_ref.at[pl.ds(ro + ln.rsub * SROW, SROW),
                               pl.ds(c * CW + ln.half * CW2, CW2)]

        def oblk(c, ln):
            return out_ref.at[pl.ds(ro + ln.rsub * SROW, SROW),
                              pl.ds(c * CW + ln.half * CW2, CW2)]

        for ln in lanes:
            pltpu.make_async_copy(blk(me, ln), ln.ld, ln.ldsem).start()
        for ln in lanes:
            pltpu.make_async_copy(blk(me, ln), ln.ld, ln.ldsem).wait()
            ln.send[0] = ln.ld[...].astype(jnp.bfloat16)
            ln.rdma = rs_rdma(ln, 0)
            ln.rdma.start()
            pltpu.make_async_copy(blk(ln.chunk(1), ln),
                                  ln.ld, ln.ldsem).start()

        for s in range(N_DEV - 1):
            slot = s % 2
            for ln in lanes:
                pltpu.make_async_copy(blk(ln.chunk(s + 1), ln),
                                      ln.ld, ln.ldsem).wait()
                ln.rdma.wait()
                if s < N_DEV - 2:
                    ln.send[(s + 1) % 2] = (
                        ln.recv[slot].astype(jnp.float32) + ln.ld[...]
                    ).astype(jnp.bfloat16)
                else:
                    own = ln.chunk(N_DEV - 1)
                    y = (ln.recv[slot].astype(jnp.float32) + ln.ld[...]
                         ) * scale_ref[0, 0]
                    sg = 1.0 / (1.0 + jnp.exp(-jnp.clip(y, -60.0, 60.0)))
                    ln.stg[1] = y * sg
                    ln.send[0] = ln.stg[1].astype(jnp.bfloat16)
                    pltpu.make_async_copy(ln.stg.at[1], oblk(own, ln),
                                          ln.stsem.at[1]).start()
                    pl.semaphore_wait(ln.agcred, 1)
                    ln.rdma = ag_rdma(ln, 0)
                    ln.rdma.start()
                if s <= N_DEV - 4:
                    pl.semaphore_signal(ln.rscred, inc=1,
                                        device_id=(ln.frm,),
                                        device_id_type=pl.DeviceIdType.MESH)
                if s == N_DEV - 3:
                    pl.semaphore_signal(ln.agcred, inc=1,
                                        device_id=(ln.frm,),
                                        device_id_type=pl.DeviceIdType.MESH)
                if s < N_DEV - 2:
                    if s + 1 >= 2:
                        pl.semaphore_wait(ln.rscred, 1)
                    ln.rdma = rs_rdma(ln, s + 1)
                    ln.rdma.start()
                    pltpu.make_async_copy(blk(ln.chunk(s + 2), ln),
                                          ln.ld, ln.ldsem).start()

        for t in range(N_DEV - 1):
            for ln in lanes:
                ln.rdma.wait()
                if t <= N_DEV - 3:
                    pl.semaphore_signal(ln.agcred, inc=1,
                                        device_id=(ln.frm,),
                                        device_id_type=pl.DeviceIdType.MESH)
                if t < N_DEV - 2:
                    pl.semaphore_wait(ln.agcred, 1)
                    ln.rdma = ag_rdma(ln, t + 1)
                    ln.rdma.start()
                r = ln.chunk(t)
                if t >= 1:
                    pltpu.make_async_copy(ln.stg.at[t % 2], oblk(r, ln),
                                          ln.stsem.at[t % 2]).wait()
                ln.stg[t % 2] = ln.send[(t + 1) % 2].astype(jnp.float32)
                pltpu.make_async_copy(ln.stg.at[t % 2], oblk(r, ln),
                                      ln.stsem.at[t % 2]).start()
        for ln in lanes:
            for sl in (1, 0):
                pltpu.make_async_copy(ln.stg.at[sl], oblk(me, ln),
                                      ln.stsem.at[sl]).wait()

    for h in range(M // HROW):
        run_pass(h * HROW)


def kernel(x, w_mat, scale_x, scale_w):
    partial = jnp.dot(x.astype(jnp.bfloat16), w_mat.astype(jnp.bfloat16),
                      preferred_element_type=jnp.float32)
    scale = jnp.reshape(scale_x.astype(jnp.float32)
                        * scale_w.astype(jnp.float32), (1, 1))

    lane_bufs = [
        pltpu.VMEM((2, SROW, CW2), jnp.bfloat16),
        pltpu.VMEM((2, SROW, CW2), jnp.bfloat16),
        pltpu.VMEM((SROW, CW2), jnp.float32),
        pltpu.VMEM((2, SROW, CW2), jnp.float32),
    ]
    lane_sems = [
        pltpu.SemaphoreType.DMA((2,)),
        pltpu.SemaphoreType.DMA((2,)),
        pltpu.SemaphoreType.DMA,
        pltpu.SemaphoreType.DMA((2,)),
        pltpu.SemaphoreType.DMA((2,)),
        pltpu.SemaphoreType.DMA((2,)),
    ]
    scratch = (lane_bufs * N_LANE
               + lane_sems * N_LANE
               + [pltpu.SemaphoreType.REGULAR] * (2 * N_LANE))

    return pl.pallas_call(
        _ar_body,
        out_shape=jax.ShapeDtypeStruct((M, N), jnp.float32),
        in_specs=[
            pl.BlockSpec(memory_space=pl.ANY),
            pl.BlockSpec(memory_space=pltpu.MemorySpace.SMEM),
        ],
        out_specs=pl.BlockSpec(memory_space=pl.ANY),
        scratch_shapes=scratch,
        compiler_params=pltpu.CompilerParams(collective_id=0),
    )(partial, scale)
